# baseline (speedup 1.0000x reference)
import functools

import jax
import jax.numpy as jnp
from jax import lax
from jax.experimental import pallas as pl
from jax.experimental.pallas import tpu as pltpu
from jax.experimental.pallas import tpu_sc as plsc

B, M, NEIGH, DIM = 10000, 1, 32, 128

TB = 512
S_SC = 5632
OFF_BLKS = S_SC // TB

NC, NS = 2, 16
NW = NC * NS
RPW = S_SC // NW
R = 4
NCHUNK = RPW // R


_DNUMS = lax.GatherDimensionNumbers(
    offset_dims=(), collapsed_slice_dims=(0,), start_index_map=(0,))


def _permute(v, idx):
    return lax.gather(v, idx[:, None], _DNUMS, (1,),
                      mode=lax.GatherScatterMode.PROMISE_IN_BOUNDS)


def _lane_allsum(v, iota16):
    for k in (1, 2, 4, 8):
        v = v + _permute(v, iota16 ^ k)
    return v


def _lane_allmax(v, iota16):
    for k in (1, 2, 4, 8):
        v = jnp.maximum(v, _permute(v, iota16 ^ k))
    return v


def _sc_weights_body(rel_hbm, ue_hbm, w_hbm, relb, ueb, wb, scores,
                     sem_r0, sem_r1, sem_u0, sem_u1):
    wid = lax.axis_index("s") * NC + lax.axis_index("c")
    base = wid * RPW
    sem_r = (sem_r0, sem_r1)
    sem_u = (sem_u0, sem_u1)

    def start(chunk, buf):
        row = base + chunk * R
        pltpu.make_async_copy(
            rel_hbm.at[pl.ds(row, R)], relb.at[buf], sem_r[buf]).start()
        pltpu.make_async_copy(
            ue_hbm.at[pl.ds(row, R)], ueb.at[buf], sem_u[buf]).start()

    def wait(chunk, buf):
        row = base + chunk * R
        pltpu.make_async_copy(
            rel_hbm.at[pl.ds(row, R)], relb.at[buf], sem_r[buf]).wait()
        pltpu.make_async_copy(
            ue_hbm.at[pl.ds(row, R)], ueb.at[buf], sem_u[buf]).wait()

    start(0, 0)
    start(1, 1)

    iota16 = lax.iota(jnp.int32, 16)

    def do_chunk(c, buf):
        wait(c, buf)
        for r in range(R):
            def nb(n, carry):
                p0 = relb[buf, r, n, pl.ds(0, 16)] * ueb[buf, r, n, pl.ds(0, 16)]
                p1 = relb[buf, r, n, pl.ds(16, 16)] * ueb[buf, r, n, pl.ds(16, 16)]
                p2 = relb[buf, r, n, pl.ds(32, 16)] * ueb[buf, r, n, pl.ds(32, 16)]
                p3 = relb[buf, r, n, pl.ds(48, 16)] * ueb[buf, r, n, pl.ds(48, 16)]
                p4 = relb[buf, r, n, pl.ds(64, 16)] * ueb[buf, r, n, pl.ds(64, 16)]
                p5 = relb[buf, r, n, pl.ds(80, 16)] * ueb[buf, r, n, pl.ds(80, 16)]
                p6 = relb[buf, r, n, pl.ds(96, 16)] * ueb[buf, r, n, pl.ds(96, 16)]
                p7 = relb[buf, r, n, pl.ds(112, 16)] * ueb[buf, r, n, pl.ds(112, 16)]
                acc = ((p0 + p1) + (p2 + p3)) + ((p4 + p5) + (p6 + p7))
                acc = _lane_allsum(acc, iota16)
                scores[pl.ds(n, 16)] = acc
                return carry
            lax.fori_loop(0, NEIGH, nb, 0, unroll=2)
            s0 = scores[pl.ds(0, 16)]
            s1 = scores[pl.ds(16, 16)]
            mx = _lane_allmax(jnp.maximum(s0, s1), iota16)
            e0 = jnp.exp(s0 - mx)
            e1 = jnp.exp(s1 - mx)
            dvec = _lane_allsum(e0 + e1, iota16)
            row = c * R + r
            wb[row, pl.ds(0, 16)] = e0 / dvec
            wb[row, pl.ds(16, 16)] = e1 / dvec
        nxt = c + 2
        @pl.when(nxt < NCHUNK)
        def _():
            start(nxt, buf)

    def body2(c2, carry):
        do_chunk(c2 * 2, 0)
        do_chunk(c2 * 2 + 1, 1)
        return carry

    lax.fori_loop(0, NCHUNK // 2, body2, 0)
    pltpu.sync_copy(wb, w_hbm.at[pl.ds(base, RPW)])


def _sc_weights(rel, ue):
    mesh = plsc.VectorSubcoreMesh(core_axis_name="c", subcore_axis_name="s")
    f = functools.partial(
        pl.kernel,
        mesh=mesh,
        out_type=jax.ShapeDtypeStruct((S_SC, NEIGH), jnp.float32),
        scratch_types=[
            pltpu.VMEM((2, R, NEIGH, DIM), jnp.float32),
            pltpu.VMEM((2, R, NEIGH, DIM), jnp.float32),
            pltpu.VMEM((RPW, NEIGH), jnp.float32),
            pltpu.VMEM((NEIGH + 16, ), jnp.float32),
            pltpu.SemaphoreType.DMA,
            pltpu.SemaphoreType.DMA,
            pltpu.SemaphoreType.DMA,
            pltpu.SemaphoreType.DMA,
        ],
    )(_sc_weights_body)
    return f(rel, ue)


def _tc_fused_body(sv_ref, nv_ref, rel_ref, ue_ref, wt_ref, bias_ref, out_ref):
    rel = rel_ref[...]
    ue = ue_ref[...]
    scores = jnp.sum(rel * ue, axis=-1)
    m = jnp.max(scores, axis=-1, keepdims=True)
    e = jnp.exp(scores - m)
    w = e / jnp.sum(e, axis=-1, keepdims=True)
    nv = nv_ref[...]
    agg = jnp.sum(w[:, :, None] * nv, axis=1)
    x = sv_ref[...] + agg
    y = jnp.dot(x, wt_ref[...], preferred_element_type=jnp.float32)
    out_ref[...] = jnp.maximum(y + bias_ref[...], 0.0)


def _tc_agg_body(sv_ref, nv_ref, w_ref, wt_ref, bias_ref, out_ref):
    w = w_ref[...]
    nv = nv_ref[...]
    agg = jnp.sum(w[:, :, None] * nv, axis=1)
    x = sv_ref[...] + agg
    y = jnp.dot(x, wt_ref[...], preferred_element_type=jnp.float32)
    out_ref[...] = jnp.maximum(y + bias_ref[...], 0.0)


@jax.jit
def kernel(self_vectors, neighbor_vectors, neighbor_relations, user_embeddings, W, b):
    nb = self_vectors.shape[0]
    sv = self_vectors.reshape(nb, DIM)
    nv = neighbor_vectors.reshape(nb, NEIGH, DIM)
    rel = neighbor_relations.reshape(nb, NEIGH, DIM)
    ue = user_embeddings.reshape(nb, NEIGH, DIM)
    wt = W.T
    bias = b.reshape(1, DIM)

    w_lo = _sc_weights(rel, ue)

    n_hi = nb - S_SC
    out_hi = pl.pallas_call(
        _tc_fused_body,
        grid=(pl.cdiv(n_hi, TB),),
        in_specs=[
            pl.BlockSpec((TB, DIM), lambda i: (i + OFF_BLKS, 0)),
            pl.BlockSpec((TB, NEIGH, DIM), lambda i: (i + OFF_BLKS, 0, 0)),
            pl.BlockSpec((TB, NEIGH, DIM), lambda i: (i + OFF_BLKS, 0, 0)),
            pl.BlockSpec((TB, NEIGH, DIM), lambda i: (i + OFF_BLKS, 0, 0)),
            pl.BlockSpec((DIM, DIM), lambda i: (0, 0)),
            pl.BlockSpec((1, DIM), lambda i: (0, 0)),
        ],
        out_specs=pl.BlockSpec((TB, DIM), lambda i: (i, 0)),
        out_shape=jax.ShapeDtypeStruct((n_hi, DIM), jnp.float32),
    )(sv, nv, rel, ue, wt, bias)

    out_lo = pl.pallas_call(
        _tc_agg_body,
        grid=(S_SC // TB,),
        in_specs=[
            pl.BlockSpec((TB, DIM), lambda i: (i, 0)),
            pl.BlockSpec((TB, NEIGH, DIM), lambda i: (i, 0, 0)),
            pl.BlockSpec((TB, NEIGH), lambda i: (i, 0)),
            pl.BlockSpec((DIM, DIM), lambda i: (0, 0)),
            pl.BlockSpec((1, DIM), lambda i: (0, 0)),
        ],
        out_specs=pl.BlockSpec((TB, DIM), lambda i: (i, 0)),
        out_shape=jax.ShapeDtypeStruct((S_SC, DIM), jnp.float32),
    )(sv, nv, w_lo, wt, bias)

    out = jnp.concatenate([out_lo, out_hi], axis=0)
    return out.reshape(nb, M, DIM)


# speedup vs baseline: 1.1957x; 1.1957x over previous
import functools

import jax
import jax.numpy as jnp
from jax import lax
from jax.experimental import pallas as pl
from jax.experimental.pallas import tpu as pltpu
from jax.experimental.pallas import tpu_sc as plsc

B, M, NEIGH, DIM = 10000, 1, 32, 128

TB = 512
S_SC = 4096
OFF_BLKS = S_SC // TB

NC, NS = 2, 16
NW = NC * NS
RPW = S_SC // NW
R = 4
NCHUNK = RPW // R


_DNUMS = lax.GatherDimensionNumbers(
    offset_dims=(), collapsed_slice_dims=(0,), start_index_map=(0,))


def _permute(v, idx):
    return lax.gather(v, idx[:, None], _DNUMS, (1,),
                      mode=lax.GatherScatterMode.PROMISE_IN_BOUNDS)


def _lane_allsum(v, iota16):
    for k in (1, 2, 4, 8):
        v = v + _permute(v, iota16 ^ k)
    return v


def _lane_allmax(v, iota16):
    for k in (1, 2, 4, 8):
        v = jnp.maximum(v, _permute(v, iota16 ^ k))
    return v


def _sc_weights_body(rel_hbm, ue_hbm, w_hbm, relb, ueb, wb, scores,
                     sem_r0, sem_r1, sem_u0, sem_u1):
    wid = lax.axis_index("s") * NC + lax.axis_index("c")
    base = wid * RPW
    sem_r = (sem_r0, sem_r1)
    sem_u = (sem_u0, sem_u1)

    def start(chunk, buf):
        row = base + chunk * R
        pltpu.make_async_copy(
            rel_hbm.at[pl.ds(row, R)], relb.at[buf], sem_r[buf]).start()
        pltpu.make_async_copy(
            ue_hbm.at[pl.ds(row, R)], ueb.at[buf], sem_u[buf]).start()

    def wait(chunk, buf):
        row = base + chunk * R
        pltpu.make_async_copy(
            rel_hbm.at[pl.ds(row, R)], relb.at[buf], sem_r[buf]).wait()
        pltpu.make_async_copy(
            ue_hbm.at[pl.ds(row, R)], ueb.at[buf], sem_u[buf]).wait()

    start(0, 0)
    start(1, 1)

    iota16 = lax.iota(jnp.int32, 16)

    def do_chunk(c, buf):
        wait(c, buf)
        for r in range(R):
            def nb(n, carry):
                p0 = relb[buf, r, n, pl.ds(0, 16)] * ueb[buf, r, n, pl.ds(0, 16)]
                p1 = relb[buf, r, n, pl.ds(16, 16)] * ueb[buf, r, n, pl.ds(16, 16)]
                p2 = relb[buf, r, n, pl.ds(32, 16)] * ueb[buf, r, n, pl.ds(32, 16)]
                p3 = relb[buf, r, n, pl.ds(48, 16)] * ueb[buf, r, n, pl.ds(48, 16)]
                p4 = relb[buf, r, n, pl.ds(64, 16)] * ueb[buf, r, n, pl.ds(64, 16)]
                p5 = relb[buf, r, n, pl.ds(80, 16)] * ueb[buf, r, n, pl.ds(80, 16)]
                p6 = relb[buf, r, n, pl.ds(96, 16)] * ueb[buf, r, n, pl.ds(96, 16)]
                p7 = relb[buf, r, n, pl.ds(112, 16)] * ueb[buf, r, n, pl.ds(112, 16)]
                acc = ((p0 + p1) + (p2 + p3)) + ((p4 + p5) + (p6 + p7))
                acc = _lane_allsum(acc, iota16)
                scores[pl.ds(n, 16)] = acc
                return carry
            lax.fori_loop(0, NEIGH, nb, 0, unroll=2)
            s0 = scores[pl.ds(0, 16)]
            s1 = scores[pl.ds(16, 16)]
            mx = _lane_allmax(jnp.maximum(s0, s1), iota16)
            e0 = jnp.exp(s0 - mx)
            e1 = jnp.exp(s1 - mx)
            dvec = _lane_allsum(e0 + e1, iota16)
            row = c * R + r
            wb[row, pl.ds(0, 16)] = e0 / dvec
            wb[row, pl.ds(16, 16)] = e1 / dvec
        nxt = c + 2
        @pl.when(nxt < NCHUNK)
        def _():
            start(nxt, buf)

    def body2(c2, carry):
        do_chunk(c2 * 2, 0)
        do_chunk(c2 * 2 + 1, 1)
        return carry

    lax.fori_loop(0, NCHUNK // 2, body2, 0)
    pltpu.sync_copy(wb, w_hbm.at[pl.ds(base, RPW)])


def _sc_weights(rel, ue):
    mesh = plsc.VectorSubcoreMesh(core_axis_name="c", subcore_axis_name="s")
    f = functools.partial(
        pl.kernel,
        mesh=mesh,
        out_type=jax.ShapeDtypeStruct((S_SC, NEIGH), jnp.float32),
        scratch_types=[
            pltpu.VMEM((2, R, NEIGH, DIM), jnp.float32),
            pltpu.VMEM((2, R, NEIGH, DIM), jnp.float32),
            pltpu.VMEM((RPW, NEIGH), jnp.float32),
            pltpu.VMEM((NEIGH + 16, ), jnp.float32),
            pltpu.SemaphoreType.DMA,
            pltpu.SemaphoreType.DMA,
            pltpu.SemaphoreType.DMA,
            pltpu.SemaphoreType.DMA,
        ],
    )(_sc_weights_body)
    return f(rel, ue)


def _tc_fused_body(sv_ref, nv_ref, rel_ref, ue_ref, wt_ref, bias_ref, out_ref):
    rel = rel_ref[...]
    ue = ue_ref[...]
    scores = jnp.sum(rel * ue, axis=-1)
    m = jnp.max(scores, axis=-1, keepdims=True)
    e = jnp.exp(scores - m)
    w = e / jnp.sum(e, axis=-1, keepdims=True)
    nv = nv_ref[...]
    agg = jnp.sum(w[:, :, None] * nv, axis=1)
    x = sv_ref[...] + agg
    y = jnp.dot(x, wt_ref[...], preferred_element_type=jnp.float32)
    out_ref[...] = jnp.maximum(y + bias_ref[...], 0.0)


def _tc_agg_body(sv_ref, nv_ref, w_ref, wt_ref, bias_ref, out_ref):
    w = w_ref[...]
    nv = nv_ref[...]
    agg = jnp.sum(w[:, :, None] * nv, axis=1)
    x = sv_ref[...] + agg
    y = jnp.dot(x, wt_ref[...], preferred_element_type=jnp.float32)
    out_ref[...] = jnp.maximum(y + bias_ref[...], 0.0)


@jax.jit
def kernel(self_vectors, neighbor_vectors, neighbor_relations, user_embeddings, W, b):
    nb = self_vectors.shape[0]
    sv = self_vectors.reshape(nb, DIM)
    nv = neighbor_vectors.reshape(nb, NEIGH, DIM)
    rel = neighbor_relations.reshape(nb, NEIGH, DIM)
    ue = user_embeddings.reshape(nb, NEIGH, DIM)
    wt = W.T
    bias = b.reshape(1, DIM)

    w_lo = _sc_weights(rel, ue)

    n_hi = nb - S_SC
    out_hi = pl.pallas_call(
        _tc_fused_body,
        grid=(pl.cdiv(n_hi, TB),),
        in_specs=[
            pl.BlockSpec((TB, DIM), lambda i: (i + OFF_BLKS, 0)),
            pl.BlockSpec((TB, NEIGH, DIM), lambda i: (i + OFF_BLKS, 0, 0)),
            pl.BlockSpec((TB, NEIGH, DIM), lambda i: (i + OFF_BLKS, 0, 0)),
            pl.BlockSpec((TB, NEIGH, DIM), lambda i: (i + OFF_BLKS, 0, 0)),
            pl.BlockSpec((DIM, DIM), lambda i: (0, 0)),
            pl.BlockSpec((1, DIM), lambda i: (0, 0)),
        ],
        out_specs=pl.BlockSpec((TB, DIM), lambda i: (i, 0)),
        out_shape=jax.ShapeDtypeStruct((n_hi, DIM), jnp.float32),
    )(sv, nv, rel, ue, wt, bias)

    out_lo = pl.pallas_call(
        _tc_agg_body,
        grid=(S_SC // TB,),
        in_specs=[
            pl.BlockSpec((TB, DIM), lambda i: (i, 0)),
            pl.BlockSpec((TB, NEIGH, DIM), lambda i: (i, 0, 0)),
            pl.BlockSpec((TB, NEIGH), lambda i: (i, 0)),
            pl.BlockSpec((DIM, DIM), lambda i: (0, 0)),
            pl.BlockSpec((1, DIM), lambda i: (0, 0)),
        ],
        out_specs=pl.BlockSpec((TB, DIM), lambda i: (i, 0)),
        out_shape=jax.ShapeDtypeStruct((S_SC, DIM), jnp.float32),
    )(sv, nv, w_lo, wt, bias)

    out = jnp.concatenate([out_lo, out_hi], axis=0)
    return out.reshape(nb, M, DIM)


# speedup vs baseline: 1.3669x; 1.1431x over previous
import functools

import jax
import jax.numpy as jnp
from jax.experimental import pallas as pl

B, M, NEIGH, DIM = 10000, 1, 32, 128
TB = 512


def _agg_kernel(sv_ref, nv_ref, rel_ref, ue_ref, wt_ref, bias_ref, out_ref):
    rel = rel_ref[...]
    ue = ue_ref[...]
    scores = jnp.sum(rel * ue, axis=-1)
    m = jnp.max(scores, axis=-1, keepdims=True)
    e = jnp.exp(scores - m)
    w = e / jnp.sum(e, axis=-1, keepdims=True)
    nv = nv_ref[...]
    agg = jnp.sum(w[:, :, None] * nv, axis=1)
    x = sv_ref[...] + agg
    y = jnp.dot(x, wt_ref[...], preferred_element_type=jnp.float32)
    out_ref[...] = jnp.maximum(y + bias_ref[...], 0.0)


@jax.jit
def kernel(self_vectors, neighbor_vectors, neighbor_relations, user_embeddings, W, b):
    nb = self_vectors.shape[0]
    sv = self_vectors.reshape(nb, DIM)
    nv = neighbor_vectors.reshape(nb, NEIGH, DIM)
    rel = neighbor_relations.reshape(nb, NEIGH, DIM)
    ue = user_embeddings.reshape(nb, NEIGH, DIM)
    wt = W.T
    bias = b.reshape(1, DIM)

    grid = (pl.cdiv(nb, TB),)
    out = pl.pallas_call(
        _agg_kernel,
        grid=grid,
        in_specs=[
            pl.BlockSpec((TB, DIM), lambda i: (i, 0)),
            pl.BlockSpec((TB, NEIGH, DIM), lambda i: (i, 0, 0)),
            pl.BlockSpec((TB, NEIGH, DIM), lambda i: (i, 0, 0)),
            pl.BlockSpec((TB, NEIGH, DIM), lambda i: (i, 0, 0)),
            pl.BlockSpec((DIM, DIM), lambda i: (0, 0)),
            pl.BlockSpec((1, DIM), lambda i: (0, 0)),
        ],
        out_specs=pl.BlockSpec((TB, DIM), lambda i: (i, 0)),
        out_shape=jax.ShapeDtypeStruct((nb, DIM), jnp.float32),
    )(sv, nv, rel, ue, wt, bias)
    return out.reshape(nb, M, DIM)
